# Initial kernel scaffold; baseline (speedup 1.0000x reference)
import jax
import jax.numpy as jnp
from jax.experimental import pallas as pl


def kernel(x, Wg, bg, Wv, bv, noise):
    raise NotImplementedError("write your pallas kernel here")


# speedup vs baseline: 4.0833x; 4.0833x over previous
import jax
import jax.numpy as jnp
from jax.experimental import pallas as pl
from jax.experimental.pallas import tpu as pltpu

_N, _D, _E = 32768, 64, 64
_BLOCK = 2048


def _gating_kernel(x_ref, w_ref, b_ref, noise_ref, probs_ref, mask_ref):
    x = x_ref[...]
    y = jnp.dot(x, w_ref[...], preferred_element_type=jnp.float32) + b_ref[...]
    logits = y[:, :_E]
    var = jax.nn.softplus(y[:, _E:])
    noisy = logits + noise_ref[...] * var
    m = jnp.max(noisy, axis=-1, keepdims=True)
    e = jnp.exp(noisy - m)
    probs_ref[...] = e * (1.0 / jnp.sum(e, axis=-1, keepdims=True))
    mask_ref[...] = jnp.ones_like(mask_ref)


def kernel(x, Wg, bg, Wv, bv, noise):
    n, d = x.shape
    e = Wg.shape[0]
    w = jnp.concatenate([Wg.T, Wv.T], axis=1)
    b = jnp.concatenate([bg, bv]).reshape(1, 2 * e)
    grid = (n // _BLOCK,)
    probs, mask = pl.pallas_call(
        _gating_kernel,
        grid=grid,
        in_specs=[
            pl.BlockSpec((_BLOCK, d), lambda i: (i, 0)),
            pl.BlockSpec((d, 2 * e), lambda i: (0, 0)),
            pl.BlockSpec((1, 2 * e), lambda i: (0, 0)),
            pl.BlockSpec((_BLOCK, e), lambda i: (i, 0)),
        ],
        out_specs=[
            pl.BlockSpec((_BLOCK, e), lambda i: (i, 0)),
            pl.BlockSpec((_BLOCK, 1), lambda i: (i, 0)),
        ],
        out_shape=[
            jax.ShapeDtypeStruct((n, e), jnp.float32),
            jax.ShapeDtypeStruct((n, 1), jnp.bool_),
        ],
    )(x, w, b, noise)
    return probs, mask


# speedup vs baseline: 4.3456x; 1.0642x over previous
import jax
import jax.numpy as jnp
from jax.experimental import pallas as pl
from jax.experimental.pallas import tpu as pltpu

_N, _D, _E = 32768, 64, 64
_BLOCK = 8192


def _gating_kernel(x_ref, w_ref, b_ref, noise_ref, probs_ref, mask_ref):
    x = x_ref[...]
    y = jnp.dot(x, w_ref[...], preferred_element_type=jnp.float32) + b_ref[...]
    logits = y[:, :_E]
    var = jax.nn.softplus(y[:, _E:])
    noisy = logits + noise_ref[...] * var
    m = jnp.max(noisy, axis=-1, keepdims=True)
    e = jnp.exp(noisy - m)
    probs_ref[...] = e * (1.0 / jnp.sum(e, axis=-1, keepdims=True))
    mask_ref[...] = jnp.ones_like(mask_ref)


def kernel(x, Wg, bg, Wv, bv, noise):
    n, d = x.shape
    e = Wg.shape[0]
    w = jnp.concatenate([Wg.T, Wv.T], axis=1)
    b = jnp.concatenate([bg, bv]).reshape(1, 2 * e)
    grid = (n // _BLOCK,)
    probs, mask = pl.pallas_call(
        _gating_kernel,
        grid=grid,
        in_specs=[
            pl.BlockSpec((_BLOCK, d), lambda i: (i, 0)),
            pl.BlockSpec((d, 2 * e), lambda i: (0, 0)),
            pl.BlockSpec((1, 2 * e), lambda i: (0, 0)),
            pl.BlockSpec((_BLOCK, e), lambda i: (i, 0)),
        ],
        out_specs=[
            pl.BlockSpec((_BLOCK, e), lambda i: (i, 0)),
            pl.BlockSpec((_BLOCK, 1), lambda i: (i, 0)),
        ],
        out_shape=[
            jax.ShapeDtypeStruct((n, e), jnp.float32),
            jax.ShapeDtypeStruct((n, 1), jnp.bool_),
        ],
    )(x, w, b, noise)
    return probs, mask
